# baseline (speedup 1.0000x reference)
import functools

import jax
import jax.numpy as jnp
from jax.experimental import pallas as pl

B, N, D, E = 2, 2048, 1024, 4
H = 2 * D
EPS = 1e-5


def _ln(v, g, b):
    mu = jnp.mean(v, axis=-1, keepdims=True)
    var = jnp.mean((v - mu) ** 2, axis=-1, keepdims=True)
    return (v - mu) * jax.lax.rsqrt(var + EPS) * g + b


def _down_kernel(xe_ref, xo_ref, pwa_ref, pwb_ref, w_ref, b_ref, g_ref,
                 beta_ref, o_ref):
    w0 = jax.nn.sigmoid(pwa_ref[...] - pwb_ref[...])
    pooled = w0 * xe_ref[...] + (1.0 - w0) * xo_ref[...]
    xd = jnp.dot(pooled.astype(jnp.bfloat16), w_ref[...],
                 preferred_element_type=jnp.float32) + b_ref[...]
    o_ref[...] = jnp.maximum(_ln(xd, g_ref[...], beta_ref[...]), 0.0)


def _moe_kernel(x_ref, rw_ref, rb_ref, w1_ref, b1_ref, w2_ref, b2_ref,
                g_ref, be_ref, o_ref):
    x = x_ref[...]
    xb = x.astype(jnp.bfloat16)
    logits = jnp.dot(xb, rw_ref[...],
                     preferred_element_type=jnp.float32) + rb_ref[...]
    m = jnp.max(logits, axis=-1, keepdims=True)
    eg = jnp.exp(logits - m)
    gates = eg / jnp.sum(eg, axis=-1, keepdims=True)
    acc = jnp.zeros_like(x)
    for e in range(E):
        h = jnp.dot(xb, w1_ref[e], preferred_element_type=jnp.float32)
        h = jnp.maximum(h + b1_ref[e], 0.0)
        ye = jnp.dot(h.astype(jnp.bfloat16), w2_ref[e],
                     preferred_element_type=jnp.float32) + b2_ref[e]
        acc += gates[:, e:e + 1] * ye
    o_ref[...] = _ln(x + acc, g_ref[...], be_ref[...])


def _up_kernel(x_ref, ske_ref, sko_ref, w1_ref, b1_ref, g1_ref, be1_ref,
               w2_ref, b2_ref, pos_ref, sw_ref, oe_ref, oo_ref):
    t = jnp.dot(x_ref[...].astype(jnp.bfloat16), w1_ref[...],
                preferred_element_type=jnp.float32) + b1_ref[...]
    t = jnp.maximum(_ln(t, g1_ref[...], be1_ref[...]), 0.0)
    t = jnp.dot(t.astype(jnp.bfloat16), w2_ref[...],
                preferred_element_type=jnp.float32) + b2_ref[...]
    sw = sw_ref[0, 0]
    oe_ref[...] = t[:, :D] + pos_ref[0:1, :] + sw * ske_ref[...]
    oo_ref[...] = t[:, D:] + pos_ref[1:2, :] + sw * sko_ref[...]


def _full(shape):
    nd = len(shape)
    return pl.BlockSpec(shape, lambda i, _nd=nd: (0,) * _nd)


def _rows(t, cols):
    return pl.BlockSpec((t, cols), lambda i: (i, 0))


def _downsample(x, p, tile):
    b, n, d = x.shape
    xe = x[:, 0::2, :].reshape(b * n // 2, d)
    xo = x[:, 1::2, :].reshape(b * n // 2, d)
    pw = p['pool_w']
    pwa = jnp.tile(pw[:, 0], (b,)).reshape(b * n // 2, 1)
    pwb = jnp.tile(pw[:, 1], (b,)).reshape(b * n // 2, 1)
    tt = b * n // 2
    grid = (tt // tile,)
    return pl.pallas_call(
        _down_kernel,
        grid=grid,
        in_specs=[_rows(tile, d), _rows(tile, d), _rows(tile, 1),
                  _rows(tile, 1), _full((d, d)), _full((1, d)),
                  _full((1, d)), _full((1, d))],
        out_specs=_rows(tile, d),
        out_shape=jax.ShapeDtypeStruct((tt, d), jnp.float32),
    )(xe, xo, pwa, pwb, p['ref_W'].astype(jnp.bfloat16),
      p['ref_b'].reshape(1, d), p['ref_g'].reshape(1, d),
      p['ref_beta'].reshape(1, d))


def _moe(x, p, tile):
    tt, d = x.shape
    grid = (tt // tile,)
    return pl.pallas_call(
        _moe_kernel,
        grid=grid,
        in_specs=[_rows(tile, d), _full((d, E)), _full((1, E)),
                  _full((E, d, H)), _full((E, 1, H)), _full((E, H, d)),
                  _full((E, 1, d)), _full((1, d)), _full((1, d))],
        out_specs=_rows(tile, d),
        out_shape=jax.ShapeDtypeStruct((tt, d), jnp.float32),
    )(x, p['rW'].astype(jnp.bfloat16), p['rb'].reshape(1, E),
      p['W1'].astype(jnp.bfloat16), p['b1'].reshape(E, 1, H),
      p['W2'].astype(jnp.bfloat16), p['b2'].reshape(E, 1, d),
      p['g'].reshape(1, d), p['be'].reshape(1, d))


def _upsample(x, ske, sko, p, sw, tile):
    tt, d = x.shape
    grid = (tt // tile,)
    out = jax.ShapeDtypeStruct((tt, d), jnp.float32)
    return pl.pallas_call(
        _up_kernel,
        grid=grid,
        in_specs=[_rows(tile, d), _rows(tile, d), _rows(tile, d),
                  _full((d, 2 * d)), _full((1, 2 * d)), _full((1, 2 * d)),
                  _full((1, 2 * d)), _full((2 * d, 2 * d)),
                  _full((1, 2 * d)), _full((2, d)), _full((1, 1))],
        out_specs=[_rows(tile, d), _rows(tile, d)],
        out_shape=[out, out],
    )(x, ske, sko, p['W1'].astype(jnp.bfloat16), p['b1'].reshape(1, 2 * d),
      p['g1'].reshape(1, 2 * d), p['be1'].reshape(1, 2 * d),
      p['W2'].astype(jnp.bfloat16), p['b2'].reshape(1, 2 * d),
      p['pos'], jnp.reshape(sw, (1, 1)))


def _interleave(oe, oo, b, n, d):
    oe = oe.reshape(b, n // 2, 1, d)
    oo = oo.reshape(b, n // 2, 1, d)
    return jnp.concatenate([oe, oo], axis=2).reshape(b, n, d)


@functools.partial(jax.jit, static_argnames=())
def kernel(x, params):
    p = params
    x1 = _downsample(x, p['down1'], 512)
    x1 = _moe(x1, p['moe1'], 512)
    x1r = x1.reshape(B, N // 2, D)
    x2 = _downsample(x1r, p['down2'], 512)
    x2 = _moe(x2, p['moe2'], 512)

    ske1 = x1r[:, 0::2, :].reshape(B * N // 4, D)
    sko1 = x1r[:, 1::2, :].reshape(B * N // 4, D)
    oe, oo = _upsample(x2, ske1, sko1, p['up1'], p['sw1'], 512)
    x3 = _interleave(oe, oo, B, N // 2, D).reshape(B * N // 2, D)
    x3 = _moe(x3, p['moe3'], 512)

    x3r = x3.reshape(B, N // 2, D)
    ske2 = x[:, 0::2, :].reshape(B * N // 2, D)
    sko2 = x[:, 1::2, :].reshape(B * N // 2, D)
    oe, oo = _upsample(x3r.reshape(B * N // 2, D), ske2, sko2,
                       p['up2'], p['sw2'], 512)
    x4 = _interleave(oe, oo, B, N, D).reshape(B * N, D)
    x4 = _moe(x4, p['moe4'], 512)
    return x4.reshape(B, N, D)


# speedup vs baseline: 1.2179x; 1.2179x over previous
import functools

import jax
import jax.numpy as jnp
from jax.experimental import pallas as pl

B, N, D, E = 2, 2048, 1024, 4
H = 2 * D
EPS = 1e-5


def _ln(v, g, b):
    mu = jnp.mean(v, axis=-1, keepdims=True)
    var = jnp.mean((v - mu) ** 2, axis=-1, keepdims=True)
    return (v - mu) * jax.lax.rsqrt(var + EPS) * g + b


def _down_kernel(xp_ref, pwa_ref, pwb_ref, w_ref, b_ref, g_ref,
                 beta_ref, o_ref):
    w0 = jax.nn.sigmoid(pwa_ref[...] - pwb_ref[...])
    xp = xp_ref[...]
    pooled = w0 * xp[:, :D] + (1.0 - w0) * xp[:, D:]
    xd = jnp.dot(pooled.astype(jnp.bfloat16), w_ref[...],
                 preferred_element_type=jnp.float32) + b_ref[...]
    o_ref[...] = jnp.maximum(_ln(xd, g_ref[...], beta_ref[...]), 0.0)


def _moe_kernel(x_ref, rw_ref, rb_ref, w1_ref, b1_ref, w2_ref, b2_ref,
                g_ref, be_ref, o_ref):
    x = x_ref[...]
    xb = x.astype(jnp.bfloat16)
    logits = jnp.dot(xb, rw_ref[...],
                     preferred_element_type=jnp.float32) + rb_ref[...]
    m = jnp.max(logits, axis=-1, keepdims=True)
    eg = jnp.exp(logits - m)
    gates = eg / jnp.sum(eg, axis=-1, keepdims=True)
    acc = jnp.zeros_like(x)
    for e in range(E):
        h = jnp.dot(xb, w1_ref[e], preferred_element_type=jnp.float32)
        h = jnp.maximum(h + b1_ref[e], 0.0)
        ye = jnp.dot(h.astype(jnp.bfloat16), w2_ref[e],
                     preferred_element_type=jnp.float32) + b2_ref[e]
        acc += gates[:, e:e + 1] * ye
    o_ref[...] = _ln(x + acc, g_ref[...], be_ref[...])


def _up_kernel(x_ref, skp_ref, w1_ref, b1_ref, g1_ref, be1_ref,
               w2_ref, b2_ref, posp_ref, sw_ref, o_ref):
    t = jnp.dot(x_ref[...].astype(jnp.bfloat16), w1_ref[...],
                preferred_element_type=jnp.float32) + b1_ref[...]
    t = jnp.maximum(_ln(t, g1_ref[...], be1_ref[...]), 0.0)
    t = jnp.dot(t.astype(jnp.bfloat16), w2_ref[...],
                preferred_element_type=jnp.float32) + b2_ref[...]
    o_ref[...] = t + posp_ref[...] + sw_ref[0, 0] * skp_ref[...]


def _full(shape):
    nd = len(shape)
    return pl.BlockSpec(shape, lambda i, _nd=nd: (0,) * _nd)


def _rows(t, cols):
    return pl.BlockSpec((t, cols), lambda i: (i, 0))


def _downsample(xp, p, tile):
    tt = xp.shape[0]
    b = B
    pw = p['pool_w']
    pwa = jnp.tile(pw[:, 0], (b,)).reshape(tt, 1)
    pwb = jnp.tile(pw[:, 1], (b,)).reshape(tt, 1)
    grid = (tt // tile,)
    return pl.pallas_call(
        _down_kernel,
        grid=grid,
        in_specs=[_rows(tile, 2 * D), _rows(tile, 1),
                  _rows(tile, 1), _full((D, D)), _full((1, D)),
                  _full((1, D)), _full((1, D))],
        out_specs=_rows(tile, D),
        out_shape=jax.ShapeDtypeStruct((tt, D), jnp.float32),
    )(xp, pwa, pwb, p['ref_W'].astype(jnp.bfloat16),
      p['ref_b'].reshape(1, D), p['ref_g'].reshape(1, D),
      p['ref_beta'].reshape(1, D))


def _moe(x, p, tile):
    tt, d = x.shape
    grid = (tt // tile,)
    return pl.pallas_call(
        _moe_kernel,
        grid=grid,
        in_specs=[_rows(tile, d), _full((d, E)), _full((1, E)),
                  _full((E, d, H)), _full((E, 1, H)), _full((E, H, d)),
                  _full((E, 1, d)), _full((1, d)), _full((1, d))],
        out_specs=_rows(tile, d),
        out_shape=jax.ShapeDtypeStruct((tt, d), jnp.float32),
    )(x, p['rW'].astype(jnp.bfloat16), p['rb'].reshape(1, E),
      p['W1'].astype(jnp.bfloat16), p['b1'].reshape(E, 1, H),
      p['W2'].astype(jnp.bfloat16), p['b2'].reshape(E, 1, d),
      p['g'].reshape(1, d), p['be'].reshape(1, d))


def _upsample(x, skp, p, sw, tile):
    tt, d = x.shape
    grid = (tt // tile,)
    return pl.pallas_call(
        _up_kernel,
        grid=grid,
        in_specs=[_rows(tile, d), _rows(tile, 2 * d),
                  _full((d, 2 * d)), _full((1, 2 * d)), _full((1, 2 * d)),
                  _full((1, 2 * d)), _full((2 * d, 2 * d)),
                  _full((1, 2 * d)), _full((1, 2 * d)), _full((1, 1))],
        out_specs=_rows(tile, 2 * d),
        out_shape=jax.ShapeDtypeStruct((tt, 2 * d), jnp.float32),
    )(x, skp, p['W1'].astype(jnp.bfloat16), p['b1'].reshape(1, 2 * d),
      p['g1'].reshape(1, 2 * d), p['be1'].reshape(1, 2 * d),
      p['W2'].astype(jnp.bfloat16), p['b2'].reshape(1, 2 * d),
      p['pos'].reshape(1, 2 * d), jnp.reshape(sw, (1, 1)))


@functools.partial(jax.jit, static_argnames=())
def kernel(x, params):
    p = params
    xp0 = x.reshape(B * N // 2, 2 * D)
    x1 = _downsample(xp0, p['down1'], 512)
    x1 = _moe(x1, p['moe1'], 512)
    xp1 = x1.reshape(B * N // 4, 2 * D)
    x2 = _downsample(xp1, p['down2'], 512)
    x2 = _moe(x2, p['moe2'], 512)

    x3p = _upsample(x2, xp1, p['up1'], p['sw1'], 512)
    x3 = _moe(x3p.reshape(B * N // 2, D), p['moe3'], 512)

    x4p = _upsample(x3, xp0, p['up2'], p['sw2'], 512)
    x4 = _moe(x4p.reshape(B * N, D), p['moe4'], 512)
    return x4.reshape(B, N, D)


# speedup vs baseline: 1.3415x; 1.1015x over previous
import functools

import jax
import jax.numpy as jnp
from jax.experimental import pallas as pl

B, N, D, E = 2, 2048, 1024, 4
H = 2 * D
EPS = 1e-5


def _ln(v, g, b):
    mu = jnp.mean(v, axis=-1, keepdims=True)
    var = jnp.mean((v - mu) ** 2, axis=-1, keepdims=True)
    return (v - mu) * jax.lax.rsqrt(var + EPS) * g + b


def _down_kernel(xp_ref, pwa_ref, pwb_ref, w_ref, b_ref, g_ref,
                 beta_ref, o_ref):
    w0 = jax.nn.sigmoid(pwa_ref[...] - pwb_ref[...])
    xp = xp_ref[...]
    pooled = w0 * xp[:, :D] + (1.0 - w0) * xp[:, D:]
    xd = jnp.dot(pooled.astype(jnp.bfloat16), w_ref[...],
                 preferred_element_type=jnp.float32) + b_ref[...]
    o_ref[...] = jnp.maximum(_ln(xd, g_ref[...], beta_ref[...]), 0.0)


def _moe_kernel(x_ref, rw_ref, rb_ref, w1_ref, b1_ref, w2_ref, b2_ref,
                g_ref, be_ref, o_ref):
    e = pl.program_id(1)
    x = x_ref[...]
    xb = x.astype(jnp.bfloat16)
    logits = jnp.dot(xb, rw_ref[...].astype(jnp.bfloat16),
                     preferred_element_type=jnp.float32) + rb_ref[...]
    m = jnp.max(logits, axis=-1, keepdims=True)
    eg = jnp.exp(logits - m)
    gates = eg / jnp.sum(eg, axis=-1, keepdims=True)
    lane = jax.lax.broadcasted_iota(jnp.int32, gates.shape, 1)
    ge = jnp.sum(jnp.where(lane == e, gates, 0.0), axis=-1, keepdims=True)
    h = jnp.dot(xb, w1_ref[0].astype(jnp.bfloat16),
                preferred_element_type=jnp.float32)
    h = jnp.maximum(h + b1_ref[0], 0.0)
    ye = jnp.dot(h.astype(jnp.bfloat16), w2_ref[0].astype(jnp.bfloat16),
                 preferred_element_type=jnp.float32) + b2_ref[0]
    contrib = ge * ye

    @pl.when(e == 0)
    def _():
        o_ref[...] = x + contrib

    @pl.when(jnp.logical_and(e > 0, e < E - 1))
    def _():
        o_ref[...] = o_ref[...] + contrib

    @pl.when(e == E - 1)
    def _():
        o_ref[...] = _ln(o_ref[...] + contrib, g_ref[...], be_ref[...])


def _up_kernel(x_ref, skp_ref, w1_ref, b1_ref, g1_ref, be1_ref,
               w2_ref, b2_ref, posp_ref, sw_ref, o_ref):
    t = jnp.dot(x_ref[...].astype(jnp.bfloat16), w1_ref[...],
                preferred_element_type=jnp.float32) + b1_ref[...]
    t = jnp.maximum(_ln(t, g1_ref[...], be1_ref[...]), 0.0)
    t = jnp.dot(t.astype(jnp.bfloat16), w2_ref[...],
                preferred_element_type=jnp.float32) + b2_ref[...]
    o_ref[...] = t + posp_ref[...] + sw_ref[0, 0] * skp_ref[...]


def _full(shape):
    nd = len(shape)
    return pl.BlockSpec(shape, lambda i, _nd=nd: (0,) * _nd)


def _rows(t, cols):
    return pl.BlockSpec((t, cols), lambda i: (i, 0))


def _downsample(xp, p, tile):
    tt = xp.shape[0]
    b = B
    pw = p['pool_w']
    pwa = jnp.tile(pw[:, 0], (b,)).reshape(tt, 1)
    pwb = jnp.tile(pw[:, 1], (b,)).reshape(tt, 1)
    grid = (tt // tile,)
    return pl.pallas_call(
        _down_kernel,
        grid=grid,
        in_specs=[_rows(tile, 2 * D), _rows(tile, 1),
                  _rows(tile, 1), _full((D, D)), _full((1, D)),
                  _full((1, D)), _full((1, D))],
        out_specs=_rows(tile, D),
        out_shape=jax.ShapeDtypeStruct((tt, D), jnp.float32),
    )(xp, pwa, pwb, p['ref_W'].astype(jnp.bfloat16),
      p['ref_b'].reshape(1, D), p['ref_g'].reshape(1, D),
      p['ref_beta'].reshape(1, D))


def _moe(x, p, tile):
    tt, d = x.shape
    grid = (tt // tile, E)
    row2 = pl.BlockSpec((tile, d), lambda i, e: (i, 0))
    f2 = lambda shape: pl.BlockSpec(shape, lambda i, e: (0,) * len(shape))
    exp3 = lambda s1, s2: pl.BlockSpec((1, s1, s2), lambda i, e: (e, 0, 0))
    return pl.pallas_call(
        _moe_kernel,
        grid=grid,
        in_specs=[row2, f2((d, E)), f2((1, E)),
                  exp3(d, H), exp3(1, H), exp3(H, d),
                  exp3(1, d), f2((1, d)), f2((1, d))],
        out_specs=row2,
        out_shape=jax.ShapeDtypeStruct((tt, d), jnp.float32),
    )(x, p['rW'], p['rb'].reshape(1, E),
      p['W1'], p['b1'].reshape(E, 1, H),
      p['W2'], p['b2'].reshape(E, 1, d),
      p['g'].reshape(1, d), p['be'].reshape(1, d))


def _upsample(x, skp, p, sw, tile):
    tt, d = x.shape
    grid = (tt // tile,)
    return pl.pallas_call(
        _up_kernel,
        grid=grid,
        in_specs=[_rows(tile, d), _rows(tile, 2 * d),
                  _full((d, 2 * d)), _full((1, 2 * d)), _full((1, 2 * d)),
                  _full((1, 2 * d)), _full((2 * d, 2 * d)),
                  _full((1, 2 * d)), _full((1, 2 * d)), _full((1, 1))],
        out_specs=_rows(tile, 2 * d),
        out_shape=jax.ShapeDtypeStruct((tt, 2 * d), jnp.float32),
    )(x, skp, p['W1'].astype(jnp.bfloat16), p['b1'].reshape(1, 2 * d),
      p['g1'].reshape(1, 2 * d), p['be1'].reshape(1, 2 * d),
      p['W2'].astype(jnp.bfloat16), p['b2'].reshape(1, 2 * d),
      p['pos'].reshape(1, 2 * d), jnp.reshape(sw, (1, 1)))


@functools.partial(jax.jit, static_argnames=())
def kernel(x, params):
    p = params
    xp0 = x.reshape(B * N // 2, 2 * D)
    x1 = _downsample(xp0, p['down1'], 512)
    x1 = _moe(x1, p['moe1'], 512)
    xp1 = x1.reshape(B * N // 4, 2 * D)
    x2 = _downsample(xp1, p['down2'], 512)
    x2 = _moe(x2, p['moe2'], 512)

    x3p = _upsample(x2, xp1, p['up1'], p['sw1'], 512)
    x3 = _moe(x3p.reshape(B * N // 2, D), p['moe3'], 512)

    x4p = _upsample(x3, xp0, p['up2'], p['sw2'], 512)
    x4 = _moe(x4p.reshape(B * N, D), p['moe4'], 512)
    return x4.reshape(B, N, D)


# speedup vs baseline: 1.4880x; 1.1092x over previous
import functools

import jax
import jax.numpy as jnp
from jax.experimental import pallas as pl
from jax.experimental.pallas import tpu as pltpu

B, N, D, E = 2, 2048, 1024, 4
H = 2 * D
EPS = 1e-5


def _ln(v, g, b):
    mu = jnp.mean(v, axis=-1, keepdims=True)
    var = jnp.mean((v - mu) ** 2, axis=-1, keepdims=True)
    return (v - mu) * jax.lax.rsqrt(var + EPS) * g + b


def _down_kernel(xp_ref, pwa_ref, pwb_ref, w_ref, b_ref, g_ref,
                 beta_ref, o_ref):
    w0 = jax.nn.sigmoid(pwa_ref[...] - pwb_ref[...])
    xp = xp_ref[...]
    pooled = w0 * xp[:, :D] + (1.0 - w0) * xp[:, D:]
    xd = jnp.dot(pooled.astype(jnp.bfloat16), w_ref[...].astype(jnp.bfloat16),
                 preferred_element_type=jnp.float32) + b_ref[...]
    o_ref[...] = jnp.maximum(_ln(xd, g_ref[...], beta_ref[...]), 0.0)


def _moe_kernel(x_ref, rw_ref, rb_ref, w1_ref, b1_ref, w2_ref, b2_ref,
                g_ref, be_ref, o_ref):
    e = pl.program_id(1)
    x = x_ref[...]
    xb = x.astype(jnp.bfloat16)
    logits = jnp.dot(xb, rw_ref[...].astype(jnp.bfloat16),
                     preferred_element_type=jnp.float32) + rb_ref[...]
    m = jnp.max(logits, axis=-1, keepdims=True)
    eg = jnp.exp(logits - m)
    gates = eg / jnp.sum(eg, axis=-1, keepdims=True)
    lane = jax.lax.broadcasted_iota(jnp.int32, gates.shape, 1)
    ge = jnp.sum(jnp.where(lane == e, gates, 0.0), axis=-1, keepdims=True)
    h2 = H // 2
    ye = b2_ref[0]
    for hh in range(2):
        w1h = w1_ref[0][:, hh * h2:(hh + 1) * h2].astype(jnp.bfloat16)
        h = jnp.dot(xb, w1h, preferred_element_type=jnp.float32)
        h = jnp.maximum(h + b1_ref[0][:, hh * h2:(hh + 1) * h2],
                        0.0).astype(jnp.bfloat16)
        w2h = w2_ref[0][hh * h2:(hh + 1) * h2, :].astype(jnp.bfloat16)
        ye = ye + jnp.dot(h, w2h, preferred_element_type=jnp.float32)
    contrib = ge * ye

    @pl.when(e == 0)
    def _():
        o_ref[...] = x + contrib

    @pl.when(jnp.logical_and(e > 0, e < E - 1))
    def _():
        o_ref[...] = o_ref[...] + contrib

    @pl.when(e == E - 1)
    def _():
        o_ref[...] = _ln(o_ref[...] + contrib, g_ref[...], be_ref[...])


def _up_kernel(x_ref, skp_ref, w1_ref, b1_ref, g1_ref, be1_ref,
               w2_ref, b2_ref, posp_ref, sw_ref, o_ref):
    t = jnp.dot(x_ref[...].astype(jnp.bfloat16),
                w1_ref[...].astype(jnp.bfloat16),
                preferred_element_type=jnp.float32) + b1_ref[...]
    t = jnp.maximum(_ln(t, g1_ref[...], be1_ref[...]), 0.0)
    tb = t.astype(jnp.bfloat16)
    base = posp_ref[...] + sw_ref[0, 0] * skp_ref[...] + b2_ref[...]
    for ch in range(2):
        w2h = w2_ref[:, ch * D:(ch + 1) * D].astype(jnp.bfloat16)
        o_ref[:, ch * D:(ch + 1) * D] = (
            jnp.dot(tb, w2h, preferred_element_type=jnp.float32)
            + base[:, ch * D:(ch + 1) * D])


def _full(shape):
    nd = len(shape)
    return pl.BlockSpec(shape, lambda i, _nd=nd: (0,) * _nd)


def _rows(t, cols):
    return pl.BlockSpec((t, cols), lambda i: (i, 0))


def _downsample(xp, p, tile):
    tt = xp.shape[0]
    b = B
    pw = p['pool_w']
    pwa = jnp.tile(pw[:, 0], (b,)).reshape(tt, 1)
    pwb = jnp.tile(pw[:, 1], (b,)).reshape(tt, 1)
    grid = (tt // tile,)
    return pl.pallas_call(
        _down_kernel,
        grid=grid,
        in_specs=[_rows(tile, 2 * D), _rows(tile, 1),
                  _rows(tile, 1), _full((D, D)), _full((1, D)),
                  _full((1, D)), _full((1, D))],
        out_specs=_rows(tile, D),
        out_shape=jax.ShapeDtypeStruct((tt, D), jnp.float32),
    )(xp, pwa, pwb, p['ref_W'],
      p['ref_b'].reshape(1, D), p['ref_g'].reshape(1, D),
      p['ref_beta'].reshape(1, D))


def _moe(x, p, tile):
    tt, d = x.shape
    grid = (tt // tile, E)
    row2 = pl.BlockSpec((tile, d), lambda i, e: (i, 0))
    f2 = lambda shape: pl.BlockSpec(shape, lambda i, e: (0,) * len(shape))
    exp3 = lambda s1, s2: pl.BlockSpec((1, s1, s2), lambda i, e: (e, 0, 0))
    return pl.pallas_call(
        _moe_kernel,
        grid=grid,
        in_specs=[row2, f2((d, E)), f2((1, E)),
                  exp3(d, H), exp3(1, H), exp3(H, d),
                  exp3(1, d), f2((1, d)), f2((1, d))],
        out_specs=row2,
        out_shape=jax.ShapeDtypeStruct((tt, d), jnp.float32),
        compiler_params=pltpu.CompilerParams(
            vmem_limit_bytes=100 * 1024 * 1024),
    )(x, p['rW'], p['rb'].reshape(1, E),
      p['W1'], p['b1'].reshape(E, 1, H),
      p['W2'], p['b2'].reshape(E, 1, d),
      p['g'].reshape(1, d), p['be'].reshape(1, d))


def _upsample(x, skp, p, sw, tile):
    tt, d = x.shape
    grid = (tt // tile,)
    return pl.pallas_call(
        _up_kernel,
        grid=grid,
        in_specs=[_rows(tile, d), _rows(tile, 2 * d),
                  _full((d, 2 * d)), _full((1, 2 * d)), _full((1, 2 * d)),
                  _full((1, 2 * d)), _full((2 * d, 2 * d)),
                  _full((1, 2 * d)), _full((1, 2 * d)), _full((1, 1))],
        out_specs=_rows(tile, 2 * d),
        out_shape=jax.ShapeDtypeStruct((tt, 2 * d), jnp.float32),
        compiler_params=pltpu.CompilerParams(
            vmem_limit_bytes=100 * 1024 * 1024),
    )(x, skp, p['W1'], p['b1'].reshape(1, 2 * d),
      p['g1'].reshape(1, 2 * d), p['be1'].reshape(1, 2 * d),
      p['W2'], p['b2'].reshape(1, 2 * d),
      p['pos'].reshape(1, 2 * d), jnp.reshape(sw, (1, 1)))


@functools.partial(jax.jit, static_argnames=())
def kernel(x, params):
    p = params
    xp0 = x.reshape(B * N // 2, 2 * D)
    x1 = _downsample(xp0, p['down1'], 512)
    x1 = _moe(x1, p['moe1'], 1024)
    xp1 = x1.reshape(B * N // 4, 2 * D)
    x2 = _downsample(xp1, p['down2'], 512)
    x2 = _moe(x2, p['moe2'], 1024)

    x3p = _upsample(x2, xp1, p['up1'], p['sw1'], 512)
    x3 = _moe(x3p.reshape(B * N // 2, D), p['moe3'], 1024)

    x4p = _upsample(x3, xp0, p['up2'], p['sw2'], 512)
    x4 = _moe(x4p.reshape(B * N, D), p['moe4'], 1024)
    return x4.reshape(B, N, D)


# speedup vs baseline: 1.5115x; 1.0158x over previous
import functools

import jax
import jax.numpy as jnp
from jax.experimental import pallas as pl
from jax.experimental.pallas import tpu as pltpu

B, N, D, E = 2, 2048, 1024, 4
H = 2 * D
EPS = 1e-5


def _ln(v, g, b):
    mu = jnp.mean(v, axis=-1, keepdims=True)
    var = jnp.mean((v - mu) ** 2, axis=-1, keepdims=True)
    return (v - mu) * jax.lax.rsqrt(var + EPS) * g + b


def _down_kernel(xn_ref, pwa_ref, pwb_ref, w_ref, b_ref, g_ref,
                 beta_ref, o_ref):
    w0 = jax.nn.sigmoid(pwa_ref[...] - pwb_ref[...])
    xg = xn_ref[...].reshape(w0.shape[0], 2, D)
    xe = xg[:, 0, :]
    xo = xg[:, 1, :]
    pooled = w0 * xe + (1.0 - w0) * xo
    xd = jnp.dot(pooled.astype(jnp.bfloat16), w_ref[...].astype(jnp.bfloat16),
                 preferred_element_type=jnp.float32) + b_ref[...]
    o_ref[...] = jnp.maximum(_ln(xd, g_ref[...], beta_ref[...]), 0.0)


def _moe_kernel(x_ref, rw_ref, rb_ref, w1_ref, b1_ref, w2_ref, b2_ref,
                g_ref, be_ref, o_ref):
    e = pl.program_id(1)
    x = x_ref[...]
    xb = x.astype(jnp.bfloat16)
    logits = jnp.dot(xb, rw_ref[...].astype(jnp.bfloat16),
                     preferred_element_type=jnp.float32) + rb_ref[...]
    m = jnp.max(logits, axis=-1, keepdims=True)
    eg = jnp.exp(logits - m)
    gates = eg / jnp.sum(eg, axis=-1, keepdims=True)
    lane = jax.lax.broadcasted_iota(jnp.int32, gates.shape, 1)
    ge = jnp.sum(jnp.where(lane == e, gates, 0.0), axis=-1, keepdims=True)
    h2 = H // 2
    ye = b2_ref[0]
    for hh in range(2):
        w1h = w1_ref[0][:, hh * h2:(hh + 1) * h2].astype(jnp.bfloat16)
        h = jnp.dot(xb, w1h, preferred_element_type=jnp.float32)
        h = jnp.maximum(h + b1_ref[0][:, hh * h2:(hh + 1) * h2],
                        0.0).astype(jnp.bfloat16)
        w2h = w2_ref[0][hh * h2:(hh + 1) * h2, :].astype(jnp.bfloat16)
        ye = ye + jnp.dot(h, w2h, preferred_element_type=jnp.float32)
    contrib = ge * ye

    @pl.when(e == 0)
    def _():
        o_ref[...] = x + contrib

    @pl.when(jnp.logical_and(e > 0, e < E - 1))
    def _():
        o_ref[...] = o_ref[...] + contrib

    @pl.when(e == E - 1)
    def _():
        o_ref[...] = _ln(o_ref[...] + contrib, g_ref[...], be_ref[...])


def _up_kernel(x_ref, skn_ref, w1_ref, b1_ref, g1_ref, be1_ref,
               w2_ref, b2_ref, posp_ref, sw_ref, o_ref):
    t = jnp.dot(x_ref[...].astype(jnp.bfloat16),
                w1_ref[...].astype(jnp.bfloat16),
                preferred_element_type=jnp.float32) + b1_ref[...]
    t = jnp.maximum(_ln(t, g1_ref[...], be1_ref[...]), 0.0)
    tb = t.astype(jnp.bfloat16)
    sw = sw_ref[0, 0]
    tt = tb.shape[0]
    ys = []
    for ch in range(2):
        w2h = w2_ref[:, ch * D:(ch + 1) * D].astype(jnp.bfloat16)
        ys.append(jnp.dot(tb, w2h, preferred_element_type=jnp.float32)
                  + b2_ref[:, ch * D:(ch + 1) * D]
                  + posp_ref[:, ch * D:(ch + 1) * D])
    y = jnp.stack(ys, axis=1).reshape(2 * tt, D)
    o_ref[...] = y + sw * skn_ref[...]


def _full(shape):
    nd = len(shape)
    return pl.BlockSpec(shape, lambda i, _nd=nd: (0,) * _nd)


def _rows(t, cols):
    return pl.BlockSpec((t, cols), lambda i: (i, 0))


def _downsample(xn, p, tile):
    tt = xn.shape[0] // 2
    b = B
    pw = p['pool_w']
    pwa = jnp.tile(pw[:, 0], (b,)).reshape(tt, 1)
    pwb = jnp.tile(pw[:, 1], (b,)).reshape(tt, 1)
    grid = (tt // tile,)
    return pl.pallas_call(
        _down_kernel,
        grid=grid,
        in_specs=[_rows(2 * tile, D), _rows(tile, 1),
                  _rows(tile, 1), _full((D, D)), _full((1, D)),
                  _full((1, D)), _full((1, D))],
        out_specs=_rows(tile, D),
        out_shape=jax.ShapeDtypeStruct((tt, D), jnp.float32),
    )(xn, pwa, pwb, p['ref_W'],
      p['ref_b'].reshape(1, D), p['ref_g'].reshape(1, D),
      p['ref_beta'].reshape(1, D))


def _moe(x, p, tile):
    tt, d = x.shape
    grid = (tt // tile, E)
    row2 = pl.BlockSpec((tile, d), lambda i, e: (i, 0))
    f2 = lambda shape: pl.BlockSpec(shape, lambda i, e: (0,) * len(shape))
    exp3 = lambda s1, s2: pl.BlockSpec((1, s1, s2), lambda i, e: (e, 0, 0))
    return pl.pallas_call(
        _moe_kernel,
        grid=grid,
        in_specs=[row2, f2((d, E)), f2((1, E)),
                  exp3(d, H), exp3(1, H), exp3(H, d),
                  exp3(1, d), f2((1, d)), f2((1, d))],
        out_specs=row2,
        out_shape=jax.ShapeDtypeStruct((tt, d), jnp.float32),
        compiler_params=pltpu.CompilerParams(
            vmem_limit_bytes=100 * 1024 * 1024),
    )(x, p['rW'], p['rb'].reshape(1, E),
      p['W1'], p['b1'].reshape(E, 1, H),
      p['W2'], p['b2'].reshape(E, 1, d),
      p['g'].reshape(1, d), p['be'].reshape(1, d))


def _upsample(x, skn, p, sw, tile):
    tt, d = x.shape
    grid = (tt // tile,)
    return pl.pallas_call(
        _up_kernel,
        grid=grid,
        in_specs=[_rows(tile, d), _rows(2 * tile, d),
                  _full((d, 2 * d)), _full((1, 2 * d)), _full((1, 2 * d)),
                  _full((1, 2 * d)), _full((2 * d, 2 * d)),
                  _full((1, 2 * d)), _full((1, 2 * d)), _full((1, 1))],
        out_specs=_rows(2 * tile, d),
        out_shape=jax.ShapeDtypeStruct((2 * tt, d), jnp.float32),
        compiler_params=pltpu.CompilerParams(
            vmem_limit_bytes=100 * 1024 * 1024),
    )(x, skn, p['W1'], p['b1'].reshape(1, 2 * d),
      p['g1'].reshape(1, 2 * d), p['be1'].reshape(1, 2 * d),
      p['W2'], p['b2'].reshape(1, 2 * d),
      p['pos'].reshape(1, 2 * d), jnp.reshape(sw, (1, 1)))


@functools.partial(jax.jit, static_argnames=())
def kernel(x, params):
    p = params
    xn = x.reshape(B * N, D)
    x1 = _downsample(xn, p['down1'], 512)
    x1 = _moe(x1, p['moe1'], 1024)
    x2 = _downsample(x1, p['down2'], 512)
    x2 = _moe(x2, p['moe2'], 1024)

    x3 = _upsample(x2, x1, p['up1'], p['sw1'], 512)
    x3 = _moe(x3, p['moe3'], 1024)

    x4 = _upsample(x3, xn, p['up2'], p['sw2'], 512)
    x4 = _moe(x4, p['moe4'], 1024)
    return x4.reshape(B, N, D)


# speedup vs baseline: 1.5361x; 1.0162x over previous
import functools

import jax
import jax.numpy as jnp
from jax.experimental import pallas as pl
from jax.experimental.pallas import tpu as pltpu

B, N, D, E = 2, 2048, 1024, 4
H = 2 * D
EPS = 1e-5


def _ln(v, g, b):
    mu = jnp.mean(v, axis=-1, keepdims=True)
    var = jnp.mean((v - mu) ** 2, axis=-1, keepdims=True)
    return (v - mu) * jax.lax.rsqrt(var + EPS) * g + b


def _down_kernel(xn_ref, pwa_ref, pwb_ref, w_ref, b_ref, g_ref,
                 beta_ref, o_ref):
    w0 = jax.nn.sigmoid(pwa_ref[...] - pwb_ref[...])
    xg = xn_ref[...].reshape(w0.shape[0], 2, D)
    xe = xg[:, 0, :]
    xo = xg[:, 1, :]
    pooled = w0 * xe + (1.0 - w0) * xo
    xd = jnp.dot(pooled.astype(jnp.bfloat16), w_ref[...].astype(jnp.bfloat16),
                 preferred_element_type=jnp.float32) + b_ref[...]
    o_ref[...] = jnp.maximum(_ln(xd, g_ref[...], beta_ref[...]), 0.0)


def _moe_kernel(x_ref, rw_ref, rb_ref, w1_ref, b1_ref, w2_ref, b2_ref,
                g_ref, be_ref, o_ref, xb_scr, gate_scr):
    e = pl.program_id(1)

    @pl.when(e == 0)
    def _():
        xc = x_ref[...].astype(jnp.bfloat16)
        xb_scr[...] = xc
        logits = jnp.dot(xc, rw_ref[...].astype(jnp.bfloat16),
                         preferred_element_type=jnp.float32) + rb_ref[...]
        m = jnp.max(logits, axis=-1, keepdims=True)
        eg = jnp.exp(logits - m)
        gate_scr[...] = eg / jnp.sum(eg, axis=-1, keepdims=True)

    xb = xb_scr[...]
    gates = gate_scr[...]
    lane = jax.lax.broadcasted_iota(jnp.int32, gates.shape, 1)
    ge = jnp.sum(jnp.where(lane == e, gates, 0.0), axis=-1, keepdims=True)
    h2 = H // 2
    ye = b2_ref[0]
    for hh in range(2):
        w1h = w1_ref[0][:, hh * h2:(hh + 1) * h2].astype(jnp.bfloat16)
        h = jnp.dot(xb, w1h, preferred_element_type=jnp.float32)
        h = jnp.maximum(h + b1_ref[0][:, hh * h2:(hh + 1) * h2],
                        0.0).astype(jnp.bfloat16)
        w2h = w2_ref[0][hh * h2:(hh + 1) * h2, :].astype(jnp.bfloat16)
        ye = ye + jnp.dot(h, w2h, preferred_element_type=jnp.float32)
    contrib = ge * ye

    @pl.when(e == 0)
    def _():
        o_ref[...] = x_ref[...] + contrib

    @pl.when(jnp.logical_and(e > 0, e < E - 1))
    def _():
        o_ref[...] = o_ref[...] + contrib

    @pl.when(e == E - 1)
    def _():
        o_ref[...] = _ln(o_ref[...] + contrib, g_ref[...], be_ref[...])


def _up_kernel(x_ref, skn_ref, w1_ref, b1_ref, g1_ref, be1_ref,
               w2_ref, b2_ref, posp_ref, sw_ref, o_ref):
    t = jnp.dot(x_ref[...].astype(jnp.bfloat16),
                w1_ref[...].astype(jnp.bfloat16),
                preferred_element_type=jnp.float32) + b1_ref[...]
    t = jnp.maximum(_ln(t, g1_ref[...], be1_ref[...]), 0.0)
    tb = t.astype(jnp.bfloat16)
    sw = sw_ref[0, 0]
    tt = tb.shape[0]
    ys = []
    for ch in range(2):
        w2h = w2_ref[:, ch * D:(ch + 1) * D].astype(jnp.bfloat16)
        ys.append(jnp.dot(tb, w2h, preferred_element_type=jnp.float32)
                  + b2_ref[:, ch * D:(ch + 1) * D]
                  + posp_ref[:, ch * D:(ch + 1) * D])
    y = jnp.stack(ys, axis=1).reshape(2 * tt, D)
    o_ref[...] = y + sw * skn_ref[...]


def _full(shape):
    nd = len(shape)
    return pl.BlockSpec(shape, lambda i, _nd=nd: (0,) * _nd)


def _rows(t, cols):
    return pl.BlockSpec((t, cols), lambda i: (i, 0))


def _downsample(xn, p, tile):
    tt = xn.shape[0] // 2
    b = B
    pw = p['pool_w']
    pwa = jnp.tile(pw[:, 0], (b,)).reshape(tt, 1)
    pwb = jnp.tile(pw[:, 1], (b,)).reshape(tt, 1)
    grid = (tt // tile,)
    return pl.pallas_call(
        _down_kernel,
        grid=grid,
        in_specs=[_rows(2 * tile, D), _rows(tile, 1),
                  _rows(tile, 1), _full((D, D)), _full((1, D)),
                  _full((1, D)), _full((1, D))],
        out_specs=_rows(tile, D),
        out_shape=jax.ShapeDtypeStruct((tt, D), jnp.float32),
    )(xn, pwa, pwb, p['ref_W'],
      p['ref_b'].reshape(1, D), p['ref_g'].reshape(1, D),
      p['ref_beta'].reshape(1, D))


def _moe(x, p, tile):
    tt, d = x.shape
    grid = (tt // tile, E)
    row2 = pl.BlockSpec((tile, d), lambda i, e: (i, 0))
    f2 = lambda shape: pl.BlockSpec(shape, lambda i, e: (0,) * len(shape))
    exp3 = lambda s1, s2: pl.BlockSpec((1, s1, s2), lambda i, e: (e, 0, 0))
    return pl.pallas_call(
        _moe_kernel,
        grid=grid,
        in_specs=[row2, f2((d, E)), f2((1, E)),
                  exp3(d, H), exp3(1, H), exp3(H, d),
                  exp3(1, d), f2((1, d)), f2((1, d))],
        out_specs=row2,
        out_shape=jax.ShapeDtypeStruct((tt, d), jnp.float32),
        scratch_shapes=[pltpu.VMEM((tile, d), jnp.bfloat16),
                        pltpu.VMEM((tile, E), jnp.float32)],
        compiler_params=pltpu.CompilerParams(
            vmem_limit_bytes=100 * 1024 * 1024),
    )(x, p['rW'], p['rb'].reshape(1, E),
      p['W1'], p['b1'].reshape(E, 1, H),
      p['W2'], p['b2'].reshape(E, 1, d),
      p['g'].reshape(1, d), p['be'].reshape(1, d))


def _upsample(x, skn, p, sw, tile):
    tt, d = x.shape
    grid = (tt // tile,)
    return pl.pallas_call(
        _up_kernel,
        grid=grid,
        in_specs=[_rows(tile, d), _rows(2 * tile, d),
                  _full((d, 2 * d)), _full((1, 2 * d)), _full((1, 2 * d)),
                  _full((1, 2 * d)), _full((2 * d, 2 * d)),
                  _full((1, 2 * d)), _full((1, 2 * d)), _full((1, 1))],
        out_specs=_rows(2 * tile, d),
        out_shape=jax.ShapeDtypeStruct((2 * tt, d), jnp.float32),
        compiler_params=pltpu.CompilerParams(
            vmem_limit_bytes=100 * 1024 * 1024),
    )(x, skn, p['W1'], p['b1'].reshape(1, 2 * d),
      p['g1'].reshape(1, 2 * d), p['be1'].reshape(1, 2 * d),
      p['W2'], p['b2'].reshape(1, 2 * d),
      p['pos'].reshape(1, 2 * d), jnp.reshape(sw, (1, 1)))


@functools.partial(jax.jit, static_argnames=())
def kernel(x, params):
    p = params
    xn = x.reshape(B * N, D)
    x1 = _downsample(xn, p['down1'], 512)
    x1 = _moe(x1, p['moe1'], 1024)
    x2 = _downsample(x1, p['down2'], 512)
    x2 = _moe(x2, p['moe2'], 1024)

    x3 = _upsample(x2, x1, p['up1'], p['sw1'], 512)
    x3 = _moe(x3, p['moe3'], 1024)

    x4 = _upsample(x3, xn, p['up2'], p['sw2'], 512)
    x4 = _moe(x4, p['moe4'], 1024)
    return x4.reshape(B, N, D)


# speedup vs baseline: 1.5416x; 1.0036x over previous
import functools

import jax
import jax.numpy as jnp
from jax.experimental import pallas as pl
from jax.experimental.pallas import tpu as pltpu

B, N, D, E = 2, 2048, 1024, 4
H = 2 * D
EPS = 1e-5


def _ln(v, g, b):
    mu = jnp.mean(v, axis=-1, keepdims=True)
    var = jnp.mean((v - mu) ** 2, axis=-1, keepdims=True)
    return (v - mu) * jax.lax.rsqrt(var + EPS) * g + b


def _down_kernel(xn_ref, pwa_ref, pwb_ref, w_ref, b_ref, g_ref,
                 beta_ref, o_ref):
    w0 = jax.nn.sigmoid(pwa_ref[...] - pwb_ref[...])
    xg = xn_ref[...].reshape(w0.shape[0], 2, D)
    xe = xg[:, 0, :]
    xo = xg[:, 1, :]
    pooled = w0 * xe + (1.0 - w0) * xo
    xd = jnp.dot(pooled.astype(jnp.bfloat16), w_ref[...].astype(jnp.bfloat16),
                 preferred_element_type=jnp.float32) + b_ref[...]
    o_ref[...] = jnp.maximum(_ln(xd, g_ref[...], beta_ref[...]), 0.0)


def _moe_kernel(x_ref, rw_ref, rb_ref, w1_ref, b1_ref, w2_ref, b2_ref,
                g_ref, be_ref, o_ref, xb_scr, gate_scr):
    e = pl.program_id(1)

    @pl.when(e == 0)
    def _():
        xc = x_ref[...].astype(jnp.bfloat16)
        xb_scr[...] = xc
        logits = jnp.dot(xc, rw_ref[...].astype(jnp.bfloat16),
                         preferred_element_type=jnp.float32) + rb_ref[...]
        m = jnp.max(logits, axis=-1, keepdims=True)
        eg = jnp.exp(logits - m)
        gate_scr[...] = eg / jnp.sum(eg, axis=-1, keepdims=True)

    xb = xb_scr[...]
    gates = gate_scr[...]
    lane = jax.lax.broadcasted_iota(jnp.int32, gates.shape, 1)
    ge = jnp.sum(jnp.where(lane == e, gates, 0.0), axis=-1, keepdims=True)
    h2 = H // 2
    ye = b2_ref[0]
    for hh in range(2):
        w1h = w1_ref[0][:, hh * h2:(hh + 1) * h2].astype(jnp.bfloat16)
        h = jnp.dot(xb, w1h, preferred_element_type=jnp.float32)
        h = jnp.maximum(h + b1_ref[0][:, hh * h2:(hh + 1) * h2],
                        0.0).astype(jnp.bfloat16)
        w2h = w2_ref[0][hh * h2:(hh + 1) * h2, :].astype(jnp.bfloat16)
        ye = ye + jnp.dot(h, w2h, preferred_element_type=jnp.float32)
    contrib = ge * ye

    @pl.when(e == 0)
    def _():
        o_ref[...] = x_ref[...] + contrib

    @pl.when(jnp.logical_and(e > 0, e < E - 1))
    def _():
        o_ref[...] = o_ref[...] + contrib

    @pl.when(e == E - 1)
    def _():
        o_ref[...] = _ln(o_ref[...] + contrib, g_ref[...], be_ref[...])


def _up_kernel(x_ref, skn_ref, w1_ref, b1_ref, g1_ref, be1_ref,
               w2_ref, b2_ref, posp_ref, sw_ref, o_ref):
    t = jnp.dot(x_ref[...].astype(jnp.bfloat16),
                w1_ref[...].astype(jnp.bfloat16),
                preferred_element_type=jnp.float32) + b1_ref[...]
    t = jnp.maximum(_ln(t, g1_ref[...], be1_ref[...]), 0.0)
    tb = t.astype(jnp.bfloat16)
    sw = sw_ref[0, 0]
    tt = tb.shape[0]
    ys = []
    for ch in range(2):
        w2h = w2_ref[:, ch * D:(ch + 1) * D].astype(jnp.bfloat16)
        ys.append(jnp.dot(tb, w2h, preferred_element_type=jnp.float32)
                  + b2_ref[:, ch * D:(ch + 1) * D]
                  + posp_ref[:, ch * D:(ch + 1) * D])
    y = jnp.stack(ys, axis=1).reshape(2 * tt, D)
    o_ref[...] = y + sw * skn_ref[...]


def _full(shape):
    nd = len(shape)
    return pl.BlockSpec(shape, lambda i, _nd=nd: (0,) * _nd)


def _rows(t, cols):
    return pl.BlockSpec((t, cols), lambda i: (i, 0))


def _downsample(xn, p, tile):
    tt = xn.shape[0] // 2
    pw = p['pool_w']
    pwa = pw[:, 0:1]
    pwb = pw[:, 1:2]
    nblk = pw.shape[0] // tile
    pwspec = pl.BlockSpec((tile, 1), lambda i, _n=nblk: (i % _n, 0))
    grid = (tt // tile,)
    return pl.pallas_call(
        _down_kernel,
        grid=grid,
        in_specs=[_rows(2 * tile, D), pwspec,
                  pwspec, _full((D, D)), _full((1, D)),
                  _full((1, D)), _full((1, D))],
        out_specs=_rows(tile, D),
        out_shape=jax.ShapeDtypeStruct((tt, D), jnp.float32),
    )(xn, pwa, pwb, p['ref_W'],
      p['ref_b'].reshape(1, D), p['ref_g'].reshape(1, D),
      p['ref_beta'].reshape(1, D))


def _moe(x, p, tile):
    tt, d = x.shape
    grid = (tt // tile, E)
    row2 = pl.BlockSpec((tile, d), lambda i, e: (i, 0))
    f2 = lambda shape: pl.BlockSpec(shape, lambda i, e: (0,) * len(shape))
    exp3 = lambda s1, s2: pl.BlockSpec((1, s1, s2), lambda i, e: (e, 0, 0))
    return pl.pallas_call(
        _moe_kernel,
        grid=grid,
        in_specs=[row2, f2((d, E)), f2((1, E)),
                  exp3(d, H), exp3(1, H), exp3(H, d),
                  exp3(1, d), f2((1, d)), f2((1, d))],
        out_specs=row2,
        out_shape=jax.ShapeDtypeStruct((tt, d), jnp.float32),
        scratch_shapes=[pltpu.VMEM((tile, d), jnp.bfloat16),
                        pltpu.VMEM((tile, E), jnp.float32)],
        compiler_params=pltpu.CompilerParams(
            vmem_limit_bytes=100 * 1024 * 1024),
    )(x, p['rW'], p['rb'].reshape(1, E),
      p['W1'], p['b1'].reshape(E, 1, H),
      p['W2'], p['b2'].reshape(E, 1, d),
      p['g'].reshape(1, d), p['be'].reshape(1, d))


def _upsample(x, skn, p, sw, tile):
    tt, d = x.shape
    grid = (tt // tile,)
    return pl.pallas_call(
        _up_kernel,
        grid=grid,
        in_specs=[_rows(tile, d), _rows(2 * tile, d),
                  _full((d, 2 * d)), _full((1, 2 * d)), _full((1, 2 * d)),
                  _full((1, 2 * d)), _full((2 * d, 2 * d)),
                  _full((1, 2 * d)), _full((1, 2 * d)), _full((1, 1))],
        out_specs=_rows(2 * tile, d),
        out_shape=jax.ShapeDtypeStruct((2 * tt, d), jnp.float32),
        compiler_params=pltpu.CompilerParams(
            vmem_limit_bytes=100 * 1024 * 1024),
    )(x, skn, p['W1'], p['b1'].reshape(1, 2 * d),
      p['g1'].reshape(1, 2 * d), p['be1'].reshape(1, 2 * d),
      p['W2'], p['b2'].reshape(1, 2 * d),
      p['pos'].reshape(1, 2 * d), jnp.reshape(sw, (1, 1)))


@functools.partial(jax.jit, static_argnames=())
def kernel(x, params):
    p = params
    xn = x.reshape(B * N, D)
    x1 = _downsample(xn, p['down1'], 512)
    x1 = _moe(x1, p['moe1'], 1024)
    x2 = _downsample(x1, p['down2'], 512)
    x2 = _moe(x2, p['moe2'], 1024)

    x3 = _upsample(x2, x1, p['up1'], p['sw1'], 512)
    x3 = _moe(x3, p['moe3'], 1024)

    x4 = _upsample(x3, xn, p['up2'], p['sw2'], 512)
    x4 = _moe(x4, p['moe4'], 1024)
    return x4.reshape(B, N, D)
